# baseline (speedup 1.0000x reference)
import functools

import jax
import jax.numpy as jnp
from jax import lax
from jax.experimental import pallas as pl
from jax.experimental.pallas import tpu as pltpu
from jax.experimental.pallas import tpu_sc as plsc

VOCAB = 100000
EMB = 64
CTX = 200
FLAT = CTX * EMB


def _gather_body(idx_ref, emb_ref, out_ref, sem):
    def start(t, carry):
        pltpu.make_async_copy(
            emb_ref.at[pl.ds(idx_ref[t], 1), :],
            out_ref.at[pl.ds(t, 1), :],
            sem).start()
        return carry

    lax.fori_loop(0, CTX, start, 0)

    def drain(t, carry):
        pltpu.make_async_copy(
            emb_ref.at[pl.ds(0, 1), :],
            out_ref.at[pl.ds(0, 1), :],
            sem).wait()
        return carry

    lax.fori_loop(0, CTX, drain, 0)


def _tc_gather(emb, idx):
    return pl.pallas_call(
        _gather_body,
        in_specs=[
            pl.BlockSpec(memory_space=pltpu.SMEM),
            pl.BlockSpec(memory_space=pltpu.MemorySpace.HBM),
        ],
        out_specs=pl.BlockSpec(memory_space=pltpu.VMEM),
        out_shape=jax.ShapeDtypeStruct((CTX, EMB), jnp.float32),
        scratch_shapes=[pltpu.SemaphoreType.DMA],
    )(idx, emb)


_KBLK = 1280
_KSTEPS = FLAT // _KBLK


def _mlp_body(x_ref, wa_ref, ba_ref, w1_ref, b1_ref, w2_ref, b2_ref,
              out_ref, acc_ref):
    j = pl.program_id(0)

    @pl.when(j == 0)
    def _():
        acc_ref[...] = jnp.zeros_like(acc_ref)

    acc_ref[...] += lax.dot_general(
        x_ref[...], wa_ref[...], (((1,), (1,)), ((), ())),
        preferred_element_type=jnp.float32)

    @pl.when(j == _KSTEPS - 1)
    def _():
        h0 = acc_ref[...] + ba_ref[...]
        h1 = jax.nn.relu(
            lax.dot_general(h0, w1_ref[...], (((1,), (1,)), ((), ())),
                            preferred_element_type=jnp.float32) + b1_ref[...])
        h2 = jax.nn.relu(
            lax.dot_general(h1, w2_ref[...], (((1,), (1,)), ((), ())),
                            preferred_element_type=jnp.float32) + b2_ref[...])
        out_ref[...] = h2


def _mlp_head(x, W_aug, b_aug, W1, b1, W2, b2):
    return pl.pallas_call(
        _mlp_body,
        grid=(_KSTEPS,),
        in_specs=[
            pl.BlockSpec((1, _KBLK), lambda j: (0, j)),
            pl.BlockSpec((512, _KBLK), lambda j: (0, j)),
            pl.BlockSpec((1, 512), lambda j: (0, 0)),
            pl.BlockSpec((128, 512), lambda j: (0, 0)),
            pl.BlockSpec((1, 128), lambda j: (0, 0)),
            pl.BlockSpec((64, 128), lambda j: (0, 0)),
            pl.BlockSpec((1, 64), lambda j: (0, 0)),
        ],
        out_specs=pl.BlockSpec((1, 64), lambda j: (0, 0)),
        out_shape=jax.ShapeDtypeStruct((1, 64), jnp.float32),
        scratch_shapes=[pltpu.VMEM((1, 512), jnp.float32)],
    )(x, W_aug, b_aug, W1, b1, W2, b2)


_VSTEPS = 10
_VBLK = VOCAB // _VSTEPS


def _vocab_body(h2_ref, w3_ref, b3_ref, out_ref):
    j = pl.program_id(0)
    row = lax.dot_general(
        h2_ref[...], w3_ref[...], (((1,), (1,)), ((), ())),
        preferred_element_type=jnp.float32)
    out_ref[pl.ds(j, 1), :] = row + b3_ref[pl.ds(j, 1), :]

    @pl.when(j == _VSTEPS - 1)
    def _():
        logits = out_ref[...]
        m = jnp.max(logits)
        lse = m + jnp.log(jnp.sum(jnp.exp(logits - m)))
        out_ref[...] = logits - lse


def _vocab_project(h2, W3, b3_blocks):
    return pl.pallas_call(
        _vocab_body,
        grid=(_VSTEPS,),
        in_specs=[
            pl.BlockSpec((1, 64), lambda j: (0, 0)),
            pl.BlockSpec((_VBLK, 64), lambda j: (j, 0)),
            pl.BlockSpec((_VSTEPS, _VBLK), lambda j: (0, 0)),
        ],
        out_specs=pl.BlockSpec((_VSTEPS, _VBLK), lambda j: (0, 0)),
        out_shape=jax.ShapeDtypeStruct((_VSTEPS, _VBLK), jnp.float32),
    )(h2, W3, b3_blocks)


def kernel(inputs, emb, W_aug, b_aug, W1, b1, W2, b2, W3, b3):
    idx = inputs.astype(jnp.int32)
    rows = _tc_gather(emb, idx)
    x = rows.reshape(1, FLAT)
    h2 = _mlp_head(x, W_aug, b_aug.reshape(1, 512), W1, b1.reshape(1, 128),
                   W2, b2.reshape(1, 64))
    logits = _vocab_project(h2, W3, b3.reshape(_VSTEPS, _VBLK))
    return logits.reshape(1, VOCAB)


# speedup vs baseline: 1.0057x; 1.0057x over previous
import jax
import jax.numpy as jnp
from jax import lax
from jax.experimental import pallas as pl
from jax.experimental.pallas import tpu as pltpu

VOCAB = 100000
EMB = 64
CTX = 200
FLAT = CTX * EMB

_KBLK = 1280
_KSTEPS = FLAT // _KBLK
_VSTEPS = 10
_VBLK = VOCAB // _VSTEPS
_GRID = _KSTEPS + _VSTEPS


def _gather_body(idx_ref, emb_ref, out_ref, sem):
    def start(t, carry):
        pltpu.make_async_copy(
            emb_ref.at[pl.ds(idx_ref[t], 1), :],
            out_ref.at[pl.ds(t, 1), :],
            sem).start()
        return carry

    lax.fori_loop(0, CTX, start, 0)

    def drain(t, carry):
        pltpu.make_async_copy(
            emb_ref.at[pl.ds(0, 1), :],
            out_ref.at[pl.ds(0, 1), :],
            sem).wait()
        return carry

    lax.fori_loop(0, CTX, drain, 0)


def _tc_gather(emb, idx):
    return pl.pallas_call(
        _gather_body,
        in_specs=[
            pl.BlockSpec(memory_space=pltpu.SMEM),
            pl.BlockSpec(memory_space=pltpu.MemorySpace.HBM),
        ],
        out_specs=pl.BlockSpec(memory_space=pltpu.VMEM),
        out_shape=jax.ShapeDtypeStruct((CTX, EMB), jnp.float32),
        scratch_shapes=[pltpu.SemaphoreType.DMA],
    )(idx, emb)


def _dense_body(x_ref, wa_ref, ba_ref, w1_ref, b1_ref, w2_ref, b2_ref,
                w3_ref, b3_ref, out_ref, acc_ref, h2_ref):
    j = pl.program_id(0)

    @pl.when(j == 0)
    def _():
        acc_ref[...] = jnp.zeros_like(acc_ref)

    @pl.when(j < _KSTEPS)
    def _():
        acc_ref[...] += lax.dot_general(
            x_ref[...], wa_ref[...], (((1,), (1,)), ((), ())),
            preferred_element_type=jnp.float32)

    @pl.when(j == _KSTEPS - 1)
    def _():
        h0 = acc_ref[...] + ba_ref[...]
        h1 = jax.nn.relu(
            lax.dot_general(h0, w1_ref[...], (((1,), (1,)), ((), ())),
                            preferred_element_type=jnp.float32) + b1_ref[...])
        h2_ref[...] = jax.nn.relu(
            lax.dot_general(h1, w2_ref[...], (((1,), (1,)), ((), ())),
                            preferred_element_type=jnp.float32) + b2_ref[...])

    @pl.when(j >= _KSTEPS)
    def _():
        v = j - _KSTEPS
        row = lax.dot_general(
            h2_ref[...], w3_ref[...], (((1,), (1,)), ((), ())),
            preferred_element_type=jnp.float32)
        out_ref[pl.ds(v, 1), :] = row + b3_ref[pl.ds(v, 1), :]

    @pl.when(j == _GRID - 1)
    def _():
        logits = out_ref[...]
        m = jnp.max(logits)
        lse = m + jnp.log(jnp.sum(jnp.exp(logits - m)))
        out_ref[...] = logits - lse


def _dense_stack(x, W_aug, b_aug, W1, b1, W2, b2, W3, b3):
    return pl.pallas_call(
        _dense_body,
        grid=(_GRID,),
        in_specs=[
            pl.BlockSpec((1, _KBLK), lambda j: (0, jnp.minimum(j, _KSTEPS - 1))),
            pl.BlockSpec((512, _KBLK), lambda j: (0, jnp.minimum(j, _KSTEPS - 1))),
            pl.BlockSpec((1, 512), lambda j: (0, 0)),
            pl.BlockSpec((128, 512), lambda j: (0, 0)),
            pl.BlockSpec((1, 128), lambda j: (0, 0)),
            pl.BlockSpec((64, 128), lambda j: (0, 0)),
            pl.BlockSpec((1, 64), lambda j: (0, 0)),
            pl.BlockSpec((_VBLK, 64), lambda j: (jnp.maximum(j - _KSTEPS, 0), 0)),
            pl.BlockSpec((_VSTEPS, _VBLK), lambda j: (0, 0)),
        ],
        out_specs=pl.BlockSpec((_VSTEPS, _VBLK), lambda j: (0, 0)),
        out_shape=jax.ShapeDtypeStruct((_VSTEPS, _VBLK), jnp.float32),
        scratch_shapes=[
            pltpu.VMEM((1, 512), jnp.float32),
            pltpu.VMEM((1, 64), jnp.float32),
        ],
    )(x, W_aug, b_aug.reshape(1, 512), W1, b1.reshape(1, 128),
      W2, b2.reshape(1, 64), W3, b3.reshape(_VSTEPS, _VBLK))


def kernel(inputs, emb, W_aug, b_aug, W1, b1, W2, b2, W3, b3):
    idx = inputs.astype(jnp.int32)
    rows = _tc_gather(emb, idx)
    x = rows.reshape(1, FLAT)
    out = _dense_stack(x, W_aug, b_aug, W1, b1, W2, b2, W3, b3)
    return out.reshape(1, VOCAB)
